# baseline (speedup 1.0000x reference)
import functools

import jax
import jax.numpy as jnp
from jax import lax
from jax.experimental import pallas as pl
from jax.experimental.pallas import tpu as pltpu
from jax.experimental.pallas import tpu_sc as plsc

M = 100000
D = 128
B = 4096
H = 512
L = 16
NC = 2
NS = 16
NW = NC * NS
BPW = B // NW
NWIN = B // L
DECAY = 0.99


def _sc_body(mem_hbm, val_hbm, idx_hbm, sidx_hbm,
             g_hbm, v_hbm, m_hbm,
             table_v, idx_v, sidx_v, buf_v,
             sidxw_v, ocw_v, ow_v, mw_v, scr16_v,
             o_sh, sem):
    c = lax.axis_index("c")
    s = lax.axis_index("s")
    wid = s * NC + c
    base = wid * BPW
    lanes = lax.iota(jnp.int32, L)

    with jax.named_scope("g_gather"):
        pltpu.sync_copy(sidx_hbm.at[pl.ds(base, BPW)], sidxw_v)
        g_copy = pltpu.async_copy(mem_hbm.at[sidxw_v], buf_v, sem)

    @pl.when(s != 0)
    def _g_wait():
        with jax.named_scope("g_wait"):
            g_copy.wait()
            pltpu.sync_copy(buf_v, g_hbm.at[pl.ds(base, BPW)])

    @pl.when(s == 0)
    def _join():
      with jax.named_scope("join"):
        pltpu.sync_copy(idx_hbm, idx_v)
        pltpu.sync_copy(sidx_hbm, sidx_v)

        U = 8
        shift1 = jnp.minimum(lanes + 1, L - 1)

        def scatter_blk(wb, carry):
            ws = [wb * U + j for j in range(U)]
            kvecs = [w * L + lanes for w in ws]
            iws = [idx_v[pl.ds(w * L, L)] for w in ws]
            sorts = [plsc.sort_key_val(iws[j] * B + kvecs[j], kvecs[j])
                     for j in range(U)]
            for j in range(U):
                scr16_v[pl.ds(j * L, L)] = sorts[j][0]
            nxts = [plsc.load_gather(scr16_v.at[pl.ds(j * L, L)], [shift1])
                    for j in range(U)]
            for j in range(U):
                ks, vs = sorts[j]
                win = jnp.logical_or(lanes == L - 1,
                                     (ks >> 12) != (nxts[j] >> 12))
                plsc.store_scatter(table_v, [ks >> 12], vs, mask=win)
            return carry

        lax.fori_loop(0, NWIN // U, scatter_blk, 0)

        def probe_blk(wb, carry):
            ws = [wb * U + j for j in range(U)]
            sws = [sidx_v[pl.ds(w * L, L)] for w in ws]
            os_ = [plsc.load_gather(table_v, [sw]) for sw in sws]
            ocs = [jnp.minimum(jnp.maximum(o, 0), B - 1) for o in os_]
            hits = [plsc.load_gather(idx_v, [oc]) for oc in ocs]
            for j in range(U):
                valid = ((os_[j] >= 0) & (os_[j] < B)
                         & (hits[j] == sws[j]))
                sidx_v[pl.ds(ws[j] * L, L)] = jnp.where(valid, os_[j], -1)
            return carry

        lax.fori_loop(0, NWIN // U, probe_blk, 0)
        pltpu.sync_copy(sidx_v, o_sh)
        with jax.named_scope("g_late"):
            g_copy.wait()
            pltpu.sync_copy(buf_v, g_hbm.at[pl.ds(base, BPW)])

    with jax.named_scope("barrier"):
        plsc.subcore_barrier()

    with jax.named_scope("v_gather"):
        pltpu.sync_copy(o_sh.at[pl.ds(base, BPW)], ow_v)

        def mask_win(w, carry):
            ow = ow_v[pl.ds(w * L, L)]
            pos = base + w * L + lanes
            ocw_v[pl.ds(w * L, L)] = jnp.where(ow >= 0, ow, pos)
            mw_v[pl.ds(w * L, L)] = jnp.where(ow >= 0, 1.0, 0.0).astype(jnp.float32)
            return carry

        lax.fori_loop(0, BPW // L, mask_win, 0)

        pltpu.async_copy(val_hbm.at[ocw_v], buf_v, sem).wait()
        pltpu.sync_copy(buf_v, v_hbm.at[pl.ds(base, BPW)])
        pltpu.sync_copy(mw_v, m_hbm.at[pl.ds(base, BPW)])


_sc_join_gather = functools.partial(
    pl.kernel,
    out_type=(
        jax.ShapeDtypeStruct((B, D), jnp.float32),
        jax.ShapeDtypeStruct((B, D), jnp.float32),
        jax.ShapeDtypeStruct((B,), jnp.float32),
    ),
    mesh=plsc.VectorSubcoreMesh(
        core_axis_name="c", subcore_axis_name="s",
        num_cores=NC, num_subcores=NS),
    scratch_types=[
        pltpu.VMEM((M,), jnp.int32),
        pltpu.VMEM((B,), jnp.int32),
        pltpu.VMEM((B,), jnp.int32),
        pltpu.VMEM((BPW, D), jnp.float32),
        pltpu.VMEM((BPW,), jnp.int32),
        pltpu.VMEM((BPW,), jnp.int32),
        pltpu.VMEM((BPW,), jnp.int32),
        pltpu.VMEM((BPW,), jnp.float32),
        pltpu.VMEM((8 * L,), jnp.int32),
        pltpu.VMEM_SHARED((B,), jnp.int32),
        pltpu.SemaphoreType.DMA,
    ],
    compiler_params=pltpu.CompilerParams(needs_layout_passes=False),
)(_sc_body)


RB = 512


def _tc_body(g_ref, v_ref, m_ref, w1_ref, b1_ref, w2_ref, b2_ref, out_ref):
    r_iota = lax.broadcasted_iota(jnp.int32, (RB, 128), 0)
    c_iota = lax.broadcasted_iota(jnp.int32, (RB, 128), 1)
    blk = lax.broadcasted_iota(jnp.int32, (RB, RB // 128), 1)
    onehot_blk = ((r_iota[:, :RB // 128] >> 7) == blk).astype(jnp.float32)
    onehot_col = ((r_iota & 127) == c_iota).astype(jnp.float32)
    m2d = m_ref[...].reshape(RB // 128, 128)
    t1 = jnp.dot(onehot_blk, m2d, preferred_element_type=jnp.float32)
    m = jnp.dot(t1 * onehot_col, jnp.ones((128, 128), jnp.float32),
                preferred_element_type=jnp.float32)
    g = g_ref[...]
    replay = DECAY * g + m * (v_ref[...] - DECAY * g)
    h = jnp.maximum(
        jnp.dot(replay, w1_ref[...], preferred_element_type=jnp.float32)
        + b1_ref[...], 0.0)
    out_ref[...] = (
        jnp.dot(h, w2_ref[...], preferred_element_type=jnp.float32)
        + b2_ref[...])


_tc_mlp = pl.pallas_call(
    _tc_body,
    grid=(B // RB,),
    in_specs=[
        pl.BlockSpec((RB, D), lambda i: (i, 0)),
        pl.BlockSpec((RB, D), lambda i: (i, 0)),
        pl.BlockSpec((1, RB // 128, 128), lambda i: (i, 0, 0)),
        pl.BlockSpec((D, H), lambda i: (0, 0)),
        pl.BlockSpec((1, H), lambda i: (0, 0)),
        pl.BlockSpec((H, D), lambda i: (0, 0)),
        pl.BlockSpec((1, D), lambda i: (0, 0)),
    ],
    out_specs=pl.BlockSpec((RB, D), lambda i: (i, 0)),
    out_shape=jax.ShapeDtypeStruct((B, D), jnp.float32),
)


def kernel(mem, val, idx, sample_idx, W1, b1, W2, b2):
    g, v, m = _sc_join_gather(mem, val, idx, sample_idx)
    return _tc_mlp(g, v, m.reshape(B // RB, RB // 128, 128),
                   W1, b1.reshape(1, H), W2, b2.reshape(1, D))


# speedup vs baseline: 1.1407x; 1.1407x over previous
import functools

import jax
import jax.numpy as jnp
from jax import lax
from jax.experimental import pallas as pl
from jax.experimental.pallas import tpu as pltpu
from jax.experimental.pallas import tpu_sc as plsc

M = 100000
D = 128
B = 4096
H = 512
L = 16
NC = 2
NS = 16
NW = NC * NS
BPW = B // NW
NWIN = B // L
DECAY = 0.99


def _sc_body(mem_hbm, val_hbm, idx_hbm, sidx_hbm,
             g_hbm, v_hbm, m_hbm,
             table_v, idx_v, sidx_v, buf_v,
             sidxw_v, ocw_v, ow_v, mw_v, scr16_v,
             o_sh, sem, sem2, sem3):
    c = lax.axis_index("c")
    s = lax.axis_index("s")
    wid = s * NC + c
    base = wid * BPW
    lanes = lax.iota(jnp.int32, L)

    @pl.when(s == 0)
    def _prefetch():
        pltpu.async_copy(idx_hbm, idx_v, sem2)
        pltpu.async_copy(sidx_hbm, sidx_v, sem3)

    with jax.named_scope("g_gather"):
        pltpu.sync_copy(sidx_hbm.at[pl.ds(base, BPW)], sidxw_v)
        g_copy = pltpu.async_copy(mem_hbm.at[sidxw_v], buf_v, sem)

    @pl.when(s != 0)
    def _g_wait():
        with jax.named_scope("g_wait"):
            g_copy.wait()
            pltpu.sync_copy(buf_v, g_hbm.at[pl.ds(base, BPW)])

    @pl.when(s == 0)
    def _join():
      with jax.named_scope("join"):
        pltpu.make_async_copy(idx_hbm, idx_v, sem2).wait()
        pltpu.make_async_copy(sidx_hbm, sidx_v, sem3).wait()

        U = 8
        shift1 = jnp.minimum(lanes + 1, L - 1)

        def scatter_blk(wb, carry):
            ws = [wb * U + j for j in range(U)]
            kvecs = [w * L + lanes for w in ws]
            iws = [idx_v[pl.ds(w * L, L)] for w in ws]
            sorts = [plsc.sort_key_val(iws[j] * B + kvecs[j], kvecs[j])
                     for j in range(U)]
            for j in range(U):
                scr16_v[pl.ds(j * L, L)] = sorts[j][0]
            nxts = [plsc.load_gather(scr16_v.at[pl.ds(j * L, L)], [shift1])
                    for j in range(U)]
            for j in range(U):
                ks, vs = sorts[j]
                win = jnp.logical_or(lanes == L - 1,
                                     (ks >> 12) != (nxts[j] >> 12))
                plsc.store_scatter(table_v, [ks >> 12], vs, mask=win)
            return carry

        lax.fori_loop(0, NWIN // U, scatter_blk, 0)

        def probe_blk(wb, carry):
            ws = [wb * U + j for j in range(U)]
            sws = [sidx_v[pl.ds(w * L, L)] for w in ws]
            os_ = [plsc.load_gather(table_v, [sw]) for sw in sws]
            ocs = [jnp.minimum(jnp.maximum(o, 0), B - 1) for o in os_]
            hits = [plsc.load_gather(idx_v, [oc]) for oc in ocs]
            for j in range(U):
                valid = ((os_[j] >= 0) & (os_[j] < B)
                         & (hits[j] == sws[j]))
                sidx_v[pl.ds(ws[j] * L, L)] = jnp.where(valid, os_[j], -1)
            return carry

        lax.fori_loop(0, NWIN // U, probe_blk, 0)
        pltpu.sync_copy(sidx_v, o_sh)
        with jax.named_scope("g_late"):
            g_copy.wait()
            pltpu.sync_copy(buf_v, g_hbm.at[pl.ds(base, BPW)])

    with jax.named_scope("barrier"):
        plsc.subcore_barrier()

    with jax.named_scope("v_gather"):
        pltpu.sync_copy(o_sh.at[pl.ds(base, BPW)], ow_v)

        def mask_win(w, carry):
            ow = ow_v[pl.ds(w * L, L)]
            pos = base + w * L + lanes
            ocw_v[pl.ds(w * L, L)] = jnp.where(ow >= 0, ow, pos)
            mw_v[pl.ds(w * L, L)] = jnp.where(ow >= 0, 1.0, 0.0).astype(jnp.float32)
            return carry

        lax.fori_loop(0, BPW // L, mask_win, 0)

        pltpu.async_copy(val_hbm.at[ocw_v], buf_v, sem).wait()
        pltpu.sync_copy(buf_v, v_hbm.at[pl.ds(base, BPW)])
        pltpu.sync_copy(mw_v, m_hbm.at[pl.ds(base, BPW)])


_sc_join_gather = functools.partial(
    pl.kernel,
    out_type=(
        jax.ShapeDtypeStruct((B, D), jnp.float32),
        jax.ShapeDtypeStruct((B, D), jnp.float32),
        jax.ShapeDtypeStruct((B,), jnp.float32),
    ),
    mesh=plsc.VectorSubcoreMesh(
        core_axis_name="c", subcore_axis_name="s",
        num_cores=NC, num_subcores=NS),
    scratch_types=[
        pltpu.VMEM((M,), jnp.int32),
        pltpu.VMEM((B,), jnp.int32),
        pltpu.VMEM((B,), jnp.int32),
        pltpu.VMEM((BPW, D), jnp.float32),
        pltpu.VMEM((BPW,), jnp.int32),
        pltpu.VMEM((BPW,), jnp.int32),
        pltpu.VMEM((BPW,), jnp.int32),
        pltpu.VMEM((BPW,), jnp.float32),
        pltpu.VMEM((8 * L,), jnp.int32),
        pltpu.VMEM_SHARED((B,), jnp.int32),
        pltpu.SemaphoreType.DMA,
        pltpu.SemaphoreType.DMA,
        pltpu.SemaphoreType.DMA,
    ],
    compiler_params=pltpu.CompilerParams(needs_layout_passes=False),
)(_sc_body)


RB = B


def _tc_body(g_ref, v_ref, m_ref, w1_ref, b1_ref, w2_ref, b2_ref, out_ref):
    r_iota = lax.broadcasted_iota(jnp.int32, (RB, 128), 0)
    c_iota = lax.broadcasted_iota(jnp.int32, (RB, 128), 1)
    blk = lax.broadcasted_iota(jnp.int32, (RB, RB // 128), 1)
    onehot_blk = ((r_iota[:, :RB // 128] >> 7) == blk).astype(jnp.float32)
    onehot_col = ((r_iota & 127) == c_iota).astype(jnp.float32)
    m2d = m_ref[...].reshape(RB // 128, 128)
    t1 = jnp.dot(onehot_blk, m2d, preferred_element_type=jnp.float32)
    m = jnp.dot(t1 * onehot_col, jnp.ones((128, 128), jnp.float32),
                preferred_element_type=jnp.float32)
    g = g_ref[...]
    replay = DECAY * g + m * (v_ref[...] - DECAY * g)
    h = jnp.maximum(
        jnp.dot(replay.astype(jnp.bfloat16),
                w1_ref[...].astype(jnp.bfloat16),
                preferred_element_type=jnp.float32)
        + b1_ref[...], 0.0)
    out_ref[...] = (
        jnp.dot(h.astype(jnp.bfloat16), w2_ref[...].astype(jnp.bfloat16),
                preferred_element_type=jnp.float32)
        + b2_ref[...])


_tc_mlp = pl.pallas_call(
    _tc_body,
    out_shape=jax.ShapeDtypeStruct((B, D), jnp.float32),
)


def kernel(mem, val, idx, sample_idx, W1, b1, W2, b2):
    g, v, m = _sc_join_gather(mem, val, idx, sample_idx)
    return _tc_mlp(g, v, m.reshape(RB // 128, 128),
                   W1, b1.reshape(1, H), W2, b2.reshape(1, D))


# speedup vs baseline: 1.1559x; 1.0133x over previous
import functools

import jax
import jax.numpy as jnp
from jax import lax
from jax.experimental import pallas as pl
from jax.experimental.pallas import tpu as pltpu
from jax.experimental.pallas import tpu_sc as plsc

M = 100000
D = 128
B = 4096
H = 512
L = 16
NC = 2
NS = 16
NW = NC * NS
BPW = B // NW
NWIN = B // L
DECAY = 0.99


def _sc_body(mem_hbm, val_hbm, idx_hbm, sidx_hbm,
             g_hbm, v_hbm, m_hbm,
             table_v, idx_v, sidx_v, buf_v,
             sidxw_v, ocw_v, ow_v, mw_v, scr16_v,
             o_sh, sem, sem2, sem3):
    c = lax.axis_index("c")
    s = lax.axis_index("s")
    wid = s * NC + c
    base = wid * BPW
    lanes = lax.iota(jnp.int32, L)

    @pl.when(s == 0)
    def _prefetch():
        pltpu.async_copy(idx_hbm, idx_v, sem2)
        pltpu.async_copy(sidx_hbm, sidx_v, sem3)

    with jax.named_scope("g_gather"):
        pltpu.sync_copy(sidx_hbm.at[pl.ds(base, BPW)], sidxw_v)
        g_copy = pltpu.async_copy(mem_hbm.at[sidxw_v], buf_v, sem)

    @pl.when(s != 0)
    def _g_wait():
        with jax.named_scope("g_wait"):
            g_copy.wait()
            pltpu.sync_copy(buf_v, g_hbm.at[pl.ds(base, BPW)])

    @pl.when(s == 0)
    def _join():
      with jax.named_scope("join"):
        pltpu.make_async_copy(idx_hbm, idx_v, sem2).wait()
        pltpu.make_async_copy(sidx_hbm, sidx_v, sem3).wait()

        U = 8
        shift1 = jnp.minimum(lanes + 1, L - 1)

        def scatter_blk(wb, carry):
            ws = [wb * U + j for j in range(U)]
            kvecs = [w * L + lanes for w in ws]
            iws = [idx_v[pl.ds(w * L, L)] for w in ws]
            sorts = [plsc.sort_key_val(iws[j] * B + kvecs[j], kvecs[j])
                     for j in range(U)]
            for j in range(U):
                scr16_v[pl.ds(j * L, L)] = sorts[j][0]
            nxts = [plsc.load_gather(scr16_v.at[pl.ds(j * L, L)], [shift1])
                    for j in range(U)]
            for j in range(U):
                ks, vs = sorts[j]
                win = jnp.logical_or(lanes == L - 1,
                                     (ks >> 12) != (nxts[j] >> 12))
                plsc.store_scatter(table_v, [ks >> 12], vs, mask=win)
            return carry

        lax.fori_loop(0, NWIN // U, scatter_blk, 0)

        def probe_blk(wb, carry):
            ws = [wb * U + j for j in range(U)]
            sws = [sidx_v[pl.ds(w * L, L)] for w in ws]
            os_ = [plsc.load_gather(table_v, [sw]) for sw in sws]
            ocs = [jnp.minimum(jnp.maximum(o, 0), B - 1) for o in os_]
            hits = [plsc.load_gather(idx_v, [oc]) for oc in ocs]
            for j in range(U):
                valid = ((os_[j] >= 0) & (os_[j] < B)
                         & (hits[j] == sws[j]))
                sidx_v[pl.ds(ws[j] * L, L)] = jnp.where(valid, os_[j], -1)
            return carry

        lax.fori_loop(0, NWIN // U, probe_blk, 0)
        pltpu.sync_copy(sidx_v, o_sh)
        with jax.named_scope("g_late"):
            g_copy.wait()
            pltpu.sync_copy(buf_v, g_hbm.at[pl.ds(base, BPW)])

    with jax.named_scope("barrier"):
        plsc.subcore_barrier()

    with jax.named_scope("v_gather"):
        pltpu.sync_copy(o_sh.at[pl.ds(base, BPW)], ow_v)

        for w in range(BPW // L):
            ow = ow_v[pl.ds(w * L, L)]
            pos = base + w * L + lanes
            ocw_v[pl.ds(w * L, L)] = jnp.where(ow >= 0, ow, pos)
            mw_v[pl.ds(w * L, L)] = jnp.where(ow >= 0, 1.0, 0.0).astype(jnp.float32)

        v_copy = pltpu.async_copy(val_hbm.at[ocw_v], buf_v, sem)
        pltpu.sync_copy(mw_v, m_hbm.at[pl.ds(base, BPW)])
        v_copy.wait()
        pltpu.sync_copy(buf_v, v_hbm.at[pl.ds(base, BPW)])


_sc_join_gather = functools.partial(
    pl.kernel,
    out_type=(
        jax.ShapeDtypeStruct((B, D), jnp.float32),
        jax.ShapeDtypeStruct((B, D), jnp.float32),
        jax.ShapeDtypeStruct((B,), jnp.float32),
    ),
    mesh=plsc.VectorSubcoreMesh(
        core_axis_name="c", subcore_axis_name="s",
        num_cores=NC, num_subcores=NS),
    scratch_types=[
        pltpu.VMEM((M,), jnp.int32),
        pltpu.VMEM((B,), jnp.int32),
        pltpu.VMEM((B,), jnp.int32),
        pltpu.VMEM((BPW, D), jnp.float32),
        pltpu.VMEM((BPW,), jnp.int32),
        pltpu.VMEM((BPW,), jnp.int32),
        pltpu.VMEM((BPW,), jnp.int32),
        pltpu.VMEM((BPW,), jnp.float32),
        pltpu.VMEM((8 * L,), jnp.int32),
        pltpu.VMEM_SHARED((B,), jnp.int32),
        pltpu.SemaphoreType.DMA,
        pltpu.SemaphoreType.DMA,
        pltpu.SemaphoreType.DMA,
    ],
    compiler_params=pltpu.CompilerParams(needs_layout_passes=False),
)(_sc_body)


RB = B // 2


def _tc_body(g_ref, v_ref, m_ref, w1_ref, b1_ref, w2_ref, b2_ref, out_ref):
    r_iota = lax.broadcasted_iota(jnp.int32, (RB, 128), 0)
    c_iota = lax.broadcasted_iota(jnp.int32, (RB, 128), 1)
    blk = lax.broadcasted_iota(jnp.int32, (RB, RB // 128), 1)
    onehot_blk = ((r_iota[:, :RB // 128] >> 7) == blk).astype(jnp.float32)
    onehot_col = ((r_iota & 127) == c_iota).astype(jnp.float32)
    m2d = m_ref[...].reshape(RB // 128, 128)
    t1 = jnp.dot(onehot_blk, m2d, preferred_element_type=jnp.float32)
    m = jnp.dot(t1 * onehot_col, jnp.ones((128, 128), jnp.float32),
                preferred_element_type=jnp.float32)
    g = g_ref[...]
    replay = DECAY * g + m * (v_ref[...] - DECAY * g)
    h = jnp.maximum(
        jnp.dot(replay.astype(jnp.bfloat16),
                w1_ref[...].astype(jnp.bfloat16),
                preferred_element_type=jnp.float32)
        + b1_ref[...], 0.0)
    out_ref[...] = (
        jnp.dot(h.astype(jnp.bfloat16), w2_ref[...].astype(jnp.bfloat16),
                preferred_element_type=jnp.float32)
        + b2_ref[...])


_tc_mlp = pl.pallas_call(
    _tc_body,
    grid=(B // RB,),
    in_specs=[
        pl.BlockSpec((RB, D), lambda i: (i, 0)),
        pl.BlockSpec((RB, D), lambda i: (i, 0)),
        pl.BlockSpec((1, RB // 128, 128), lambda i: (i, 0, 0)),
        pl.BlockSpec((D, H), lambda i: (0, 0)),
        pl.BlockSpec((1, H), lambda i: (0, 0)),
        pl.BlockSpec((H, D), lambda i: (0, 0)),
        pl.BlockSpec((1, D), lambda i: (0, 0)),
    ],
    out_specs=pl.BlockSpec((RB, D), lambda i: (i, 0)),
    out_shape=jax.ShapeDtypeStruct((B, D), jnp.float32),
)


def kernel(mem, val, idx, sample_idx, W1, b1, W2, b2):
    g, v, m = _sc_join_gather(mem, val, idx, sample_idx)
    return _tc_mlp(g, v, m.reshape(B // RB, RB // 128, 128),
                   W1, b1.reshape(1, H), W2, b2.reshape(1, D))


# speedup vs baseline: 1.1684x; 1.0108x over previous
import functools

import jax
import jax.numpy as jnp
from jax import lax
from jax.experimental import pallas as pl
from jax.experimental.pallas import tpu as pltpu
from jax.experimental.pallas import tpu_sc as plsc

M = 100000
D = 128
B = 4096
H = 512
L = 16
NC = 2
NS = 16
NW = NC * NS
BPW = B // NW
NWIN = B // L
DECAY = 0.99


def _sc_body(mem_hbm, val_hbm, idx_hbm, sidx_hbm,
             g_hbm, v_hbm, m_hbm,
             table_v, idx_v, sidx_v, buf_v,
             sidxw_v, ocw_v, ow_v, mw_v, scr16_v,
             o_sh, sem, sem2, sem3, sem4):
    c = lax.axis_index("c")
    s = lax.axis_index("s")
    wid = s * NC + c
    base = wid * BPW
    lanes = lax.iota(jnp.int32, L)

    @pl.when(s == 0)
    def _prefetch():
        pltpu.async_copy(idx_hbm, idx_v, sem2)
        pltpu.async_copy(sidx_hbm, sidx_v, sem3)

    with jax.named_scope("g_gather"):
        pltpu.sync_copy(sidx_hbm.at[pl.ds(base, BPW)], sidxw_v)
        g_copy = pltpu.async_copy(mem_hbm.at[sidxw_v], buf_v, sem)

    @pl.when(s != 0)
    def _g_wait():
        with jax.named_scope("g_wait"):
            g_copy.wait()
            pltpu.async_copy(buf_v, g_hbm.at[pl.ds(base, BPW)], sem4)

    @pl.when(s == 0)
    def _join():
      with jax.named_scope("join"):
        pltpu.make_async_copy(idx_hbm, idx_v, sem2).wait()
        pltpu.make_async_copy(sidx_hbm, sidx_v, sem3).wait()

        U = 8
        shift1 = jnp.minimum(lanes + 1, L - 1)

        def scatter_blk(wb, carry):
            ws = [wb * U + j for j in range(U)]
            kvecs = [w * L + lanes for w in ws]
            iws = [idx_v[pl.ds(w * L, L)] for w in ws]
            sorts = [plsc.sort_key_val(iws[j] * B + kvecs[j], kvecs[j])
                     for j in range(U)]
            for j in range(U):
                scr16_v[pl.ds(j * L, L)] = sorts[j][0]
            nxts = [plsc.load_gather(scr16_v.at[pl.ds(j * L, L)], [shift1])
                    for j in range(U)]
            for j in range(U):
                ks, vs = sorts[j]
                win = jnp.logical_or(lanes == L - 1,
                                     (ks >> 12) != (nxts[j] >> 12))
                plsc.store_scatter(table_v, [ks >> 12], vs, mask=win)
            return carry

        lax.fori_loop(0, NWIN // U, scatter_blk, 0)

        def probe_blk(wb, carry):
            ws = [wb * U + j for j in range(U)]
            sws = [sidx_v[pl.ds(w * L, L)] for w in ws]
            os_ = [plsc.load_gather(table_v, [sw]) for sw in sws]
            ocs = [jnp.minimum(jnp.maximum(o, 0), B - 1) for o in os_]
            hits = [plsc.load_gather(idx_v, [oc]) for oc in ocs]
            for j in range(U):
                valid = ((os_[j] >= 0) & (os_[j] < B)
                         & (hits[j] == sws[j]))
                sidx_v[pl.ds(ws[j] * L, L)] = jnp.where(valid, os_[j], -1)
            return carry

        with jax.named_scope("g_late"):
            g_copy.wait()
            pltpu.async_copy(buf_v, g_hbm.at[pl.ds(base, BPW)], sem4)
        lax.fori_loop(0, NWIN // U, probe_blk, 0)
        pltpu.sync_copy(sidx_v, o_sh)

    with jax.named_scope("barrier"):
        plsc.subcore_barrier()

    with jax.named_scope("v_gather"):
        pltpu.sync_copy(o_sh.at[pl.ds(base, BPW)], ow_v)

        for w in range(BPW // L):
            ow = ow_v[pl.ds(w * L, L)]
            pos = base + w * L + lanes
            ocw_v[pl.ds(w * L, L)] = jnp.where(ow >= 0, ow, pos)
            mw_v[pl.ds(w * L, L)] = jnp.where(ow >= 0, 1.0, 0.0).astype(jnp.float32)

        pltpu.make_async_copy(
            buf_v, g_hbm.at[pl.ds(base, BPW)], sem4).wait()
        HB = BPW // 2
        c0 = pltpu.async_copy(
            val_hbm.at[ocw_v.at[pl.ds(0, HB)]], buf_v.at[pl.ds(0, HB)], sem)
        c1 = pltpu.async_copy(
            val_hbm.at[ocw_v.at[pl.ds(HB, HB)]],
            buf_v.at[pl.ds(HB, HB)], sem2)
        pltpu.sync_copy(mw_v, m_hbm.at[pl.ds(base, BPW)])
        c0.wait()
        f0 = pltpu.async_copy(
            buf_v.at[pl.ds(0, HB)], v_hbm.at[pl.ds(base, HB)], sem3)
        c1.wait()
        f1 = pltpu.async_copy(
            buf_v.at[pl.ds(HB, HB)], v_hbm.at[pl.ds(base + HB, HB)], sem4)
        f0.wait()
        f1.wait()


_sc_join_gather = functools.partial(
    pl.kernel,
    out_type=(
        jax.ShapeDtypeStruct((B, D), jnp.float32),
        jax.ShapeDtypeStruct((B, D), jnp.float32),
        jax.ShapeDtypeStruct((B,), jnp.float32),
    ),
    mesh=plsc.VectorSubcoreMesh(
        core_axis_name="c", subcore_axis_name="s",
        num_cores=NC, num_subcores=NS),
    scratch_types=[
        pltpu.VMEM((M,), jnp.int32),
        pltpu.VMEM((B,), jnp.int32),
        pltpu.VMEM((B,), jnp.int32),
        pltpu.VMEM((BPW, D), jnp.float32),
        pltpu.VMEM((BPW,), jnp.int32),
        pltpu.VMEM((BPW,), jnp.int32),
        pltpu.VMEM((BPW,), jnp.int32),
        pltpu.VMEM((BPW,), jnp.float32),
        pltpu.VMEM((8 * L,), jnp.int32),
        pltpu.VMEM_SHARED((B,), jnp.int32),
        pltpu.SemaphoreType.DMA,
        pltpu.SemaphoreType.DMA,
        pltpu.SemaphoreType.DMA,
        pltpu.SemaphoreType.DMA,
    ],
    compiler_params=pltpu.CompilerParams(needs_layout_passes=False),
)(_sc_body)


RB = B // 2


def _tc_body(g_ref, v_ref, m_ref, w1_ref, b1_ref, w2_ref, b2_ref, out_ref):
    r_iota = lax.broadcasted_iota(jnp.int32, (RB, 128), 0)
    c_iota = lax.broadcasted_iota(jnp.int32, (RB, 128), 1)
    blk = lax.broadcasted_iota(jnp.int32, (RB, RB // 128), 1)
    onehot_blk = ((r_iota[:, :RB // 128] >> 7) == blk).astype(jnp.float32)
    onehot_col = ((r_iota & 127) == c_iota).astype(jnp.float32)
    m2d = m_ref[...].reshape(RB // 128, 128)
    t1 = jnp.dot(onehot_blk, m2d, preferred_element_type=jnp.float32)
    m = jnp.dot(t1 * onehot_col, jnp.ones((128, 128), jnp.float32),
                preferred_element_type=jnp.float32)
    g = g_ref[...]
    replay = DECAY * g + m * (v_ref[...] - DECAY * g)
    h = jnp.maximum(
        jnp.dot(replay.astype(jnp.bfloat16),
                w1_ref[...].astype(jnp.bfloat16),
                preferred_element_type=jnp.float32)
        + b1_ref[...], 0.0)
    out_ref[...] = (
        jnp.dot(h.astype(jnp.bfloat16), w2_ref[...].astype(jnp.bfloat16),
                preferred_element_type=jnp.float32)
        + b2_ref[...])


_tc_mlp = pl.pallas_call(
    _tc_body,
    grid=(B // RB,),
    in_specs=[
        pl.BlockSpec((RB, D), lambda i: (i, 0)),
        pl.BlockSpec((RB, D), lambda i: (i, 0)),
        pl.BlockSpec((1, RB // 128, 128), lambda i: (i, 0, 0)),
        pl.BlockSpec((D, H), lambda i: (0, 0)),
        pl.BlockSpec((1, H), lambda i: (0, 0)),
        pl.BlockSpec((H, D), lambda i: (0, 0)),
        pl.BlockSpec((1, D), lambda i: (0, 0)),
    ],
    out_specs=pl.BlockSpec((RB, D), lambda i: (i, 0)),
    out_shape=jax.ShapeDtypeStruct((B, D), jnp.float32),
)


def kernel(mem, val, idx, sample_idx, W1, b1, W2, b2):
    g, v, m = _sc_join_gather(mem, val, idx, sample_idx)
    return _tc_mlp(g, v, m.reshape(B // RB, RB // 128, 128),
                   W1, b1.reshape(1, H), W2, b2.reshape(1, D))
